# baseline (speedup 1.0000x reference)
import functools

import jax
import jax.numpy as jnp
from jax import lax
from jax.experimental import pallas as pl
from jax.experimental.pallas import tpu as pltpu
from jax.experimental.pallas import tpu_sc as plsc

N = 10000
E = 160000
F_IN = 256
H = 256
LEV = 2
R_ = 3
NS = (R_ - 1) * LEV + 1
B = 32
NNZ = 800000
C = 10
NPAD = 10240


def _i0():
    return jnp.asarray(0, jnp.int32)


def _mm_body(x_ref, w_ref, o_ref):
    o_ref[...] = jnp.dot(x_ref[...], w_ref[...],
                         preferred_element_type=jnp.float32)


def _matmul(x, w, bm=2048):
    M, K = x.shape
    F = w.shape[1]
    assert M % bm == 0
    return pl.pallas_call(
        _mm_body,
        grid=(M // bm,),
        in_specs=[pl.BlockSpec((bm, K), lambda i: (i, _i0())),
                  pl.BlockSpec((K, F), lambda i: (_i0(), _i0())),
        ],
        out_specs=pl.BlockSpec((bm, F), lambda i: (i, _i0())),
        out_shape=jax.ShapeDtypeStruct((M, F), jnp.float32),
    )(x, w)


def _head_body(pm_ref, xc_ref, wf1_ref, bf1_ref, wf2_ref, bf2_ref,
               wf3_ref, bf3_ref, o_ref):
    pooled = jnp.dot(pm_ref[...], xc_ref[...],
                     preferred_element_type=jnp.float32)
    g = pooled.reshape(B, NS * 3 * H)
    h = jax.nn.relu(jnp.dot(g, wf1_ref[...],
                            preferred_element_type=jnp.float32) + bf1_ref[...])
    h = jax.nn.relu(jnp.dot(h, wf2_ref[...],
                            preferred_element_type=jnp.float32) + bf2_ref[...])
    logits = jnp.dot(h, wf3_ref[...],
                     preferred_element_type=jnp.float32) + bf3_ref[...]
    colid = jax.lax.broadcasted_iota(jnp.int32, logits.shape, 1)
    masked = jnp.where(colid < C, logits, -jnp.inf)
    mx = jnp.max(masked, axis=-1, keepdims=True)
    lse = jnp.log(jnp.sum(jnp.where(colid < C, jnp.exp(masked - mx), 0.0),
                          axis=-1, keepdims=True)) + mx
    o_ref[...] = jnp.where(colid < C, masked - lse, 0.0)


def _head(pmat, xc, wf1, bf1, wf2, bf2, wf3p, bf3p):
    full = lambda shape: pl.BlockSpec(shape, lambda: tuple(_i0() for _ in shape))
    return pl.pallas_call(
        _head_body,
        in_specs=[full((B * NS, NPAD)), full((NPAD, 3 * H)),
                  full((NS * 3 * H, 3 * H)), full((3 * H,)),
                  full((3 * H, H)), full((H,)),
                  full((H, 128)), full((128,))],
        out_specs=full((B, 128)),
        out_shape=jax.ShapeDtypeStruct((B, 128), jnp.float32),
    )(pmat, xc, wf1, bf1, wf2, bf2, wf3p, bf3p)


_SC_MESH = plsc.VectorSubcoreMesh(core_axis_name="c", subcore_axis_name="s")
PB_CH = 2000
PROWS = 5


def _c(v):
    return jnp.asarray(v, jnp.int32)


def _fori(n, body, init=0):
    return lax.fori_loop(_c(0), _c(n), body, init)


def _cv(v):
    return jnp.full((16,), v, jnp.int32)


def _divmod_n(v):
    q = (jnp.where(v >= _cv(N), _cv(1), _cv(0))
         + jnp.where(v >= _cv(2 * N), _cv(1), _cv(0))
         + jnp.where(v >= _cv(3 * N), _cv(1), _cv(0))
         + jnp.where(v >= _cv(4 * N), _cv(1), _cv(0)))
    return q, v - q * _cv(N)


def _pbuild_body(rows_hbm, cols_hbm, d_hbm, batch_hbm, p_hbm,
                 rows_v, cols_v, d_v, batch_v, acc_v):
    wid = lax.axis_index("s") * _c(2) + lax.axis_index("c")
    lo = wid * _c(PROWS)

    def zrow(r, carry):
        def zcol(j, c2):
            acc_v[r, pl.ds(j * _c(16), 16)] = jnp.zeros((16,), jnp.float32)
            return c2
        return _fori(NPAD // 16, zcol, carry)
    _fori(8, zrow, 0)

    pltpu.sync_copy(batch_hbm, batch_v)

    def chunk(ci, carry):
        base = ci * _c(PB_CH)
        pltpu.sync_copy(rows_hbm.at[pl.ds(base, PB_CH)], rows_v)
        pltpu.sync_copy(cols_hbm.at[pl.ds(base, PB_CH)], cols_v)
        pltpu.sync_copy(d_hbm.at[pl.ds(base, PB_CH)], d_v)

        def inner(k, c2):
            off = k * _c(16)
            rv = rows_v[pl.ds(off, 16)]
            cv = cols_v[pl.ds(off, 16)]
            dv = d_v[pl.ds(off, 16)]
            s, n_ = _divmod_n(rv)
            _, c = _divmod_n(cv)
            b = plsc.load_gather(batch_v, [n_])
            local = b * _cv(NS) + s - jnp.broadcast_to(lo, (16,))
            msk = (local >= _cv(0)) & (local < _cv(PROWS))
            local = jnp.where(msk, local, _cv(0))
            plsc.addupdate_scatter(acc_v, [local, c], dv, mask=msk)
            return c2
        return _fori(PB_CH // 16, inner, carry)
    _fori(NNZ // PB_CH, chunk, 0)

    pltpu.sync_copy(acc_v, p_hbm.at[wid])


def _pbuild(rows, cols, dvals, batchp):
    f = pl.kernel(
        _pbuild_body,
        out_type=jax.ShapeDtypeStruct((32, 8, NPAD), jnp.float32),
        mesh=_SC_MESH,
        compiler_params=pltpu.CompilerParams(needs_layout_passes=False),
        scratch_types=[
            pltpu.VMEM((PB_CH,), jnp.int32),
            pltpu.VMEM((PB_CH,), jnp.int32),
            pltpu.VMEM((PB_CH,), jnp.float32),
            pltpu.VMEM((NPAD,), jnp.int32),
            pltpu.VMEM((8, NPAD), jnp.float32),
        ],
    )
    out3 = f(rows, cols, dvals, batchp)
    return out3[:, :PROWS, :].reshape(B * NS, NPAD)


EC = 2000
ROWS_T = NPAD // 32


def _iota16():
    return lax.iota(jnp.int32, 16)


def _deg_body(dst_hbm, degp_hbm, dst_v, deg_v):
    wid = lax.axis_index("s") * _c(2) + lax.axis_index("c")

    def zcol(j, c2):
        deg_v[0, pl.ds(j * _c(16), 16)] = jnp.zeros((16,), jnp.float32)
        return c2
    _fori(NPAD // 16, zcol, 0)

    nch = E // EC

    def chunk(i, c2):
        ci = wid + i * _c(32)

        @pl.when(ci < _c(nch))
        def _():
            pltpu.sync_copy(dst_hbm.at[pl.ds(ci * _c(EC), EC)], dst_v)

            def inner(k, c3):
                tv = dst_v[pl.ds(k * _c(16), 16)]
                plsc.addupdate_scatter(deg_v, [_cv(0), tv],
                                       jnp.full((16,), 1.0, jnp.float32))
                return c3
            _fori(EC // 16, inner, 0)
        return c2
    _fori((nch + 31) // 32, chunk, 0)
    pltpu.sync_copy(deg_v, degp_hbm.at[wid])


def _deg(dst):
    f = pl.kernel(
        _deg_body,
        out_type=jax.ShapeDtypeStruct((32, 1, NPAD), jnp.float32),
        mesh=_SC_MESH,
        compiler_params=pltpu.CompilerParams(needs_layout_passes=False),
        scratch_types=[
            pltpu.VMEM((EC,), jnp.int32),
            pltpu.VMEM((1, NPAD), jnp.float32),
        ],
    )
    return f(dst)


def _dinv_body(degp_ref, o_ref):
    deg = jnp.sum(degp_ref[...], axis=0) + 1.0
    o_ref[...] = lax.rsqrt(jnp.maximum(deg, 1.0))


def _dinv(degp):
    full = lambda shape: pl.BlockSpec(shape, lambda: tuple(_i0() for _ in shape))
    return pl.pallas_call(
        _dinv_body,
        in_specs=[full((32, NPAD))],
        out_specs=full((NPAD,)),
        out_shape=jax.ShapeDtypeStruct((NPAD,), jnp.float32),
    )(degp)


def _spmm_body(hw_hbm, src_hbm, dst_hbm, dinv_hbm, bias_hbm, out_hbm,
               dinv_v, src_v, dst_v, esrc_l, edloc_l, enrm_l,
               stage_v, acc_v, bias_v, sem):
    wid = lax.axis_index("s") * _c(2) + lax.axis_index("c")
    lo_node = wid * _c(ROWS_T)

    pltpu.sync_copy(dinv_hbm, dinv_v)
    pltpu.sync_copy(bias_hbm, bias_v)

    def zl(k, c2):
        esrc_l[pl.ds(k * _c(16), 16)] = jnp.zeros((16,), jnp.int32)
        return c2
    _fori(EC // 16, zl, 0)

    pltpu.sync_copy(hw_hbm.at[pl.ds(lo_node, ROWS_T)], acc_v)

    def initg(g, c2):
        rows = g * _c(16) + _iota16()
        dv = plsc.load_gather(dinv_v, [jnp.broadcast_to(lo_node, (16,)) + rows])
        dv2 = dv * dv

        def initj(jj, c3):
            for u in range(8):
                jv = jnp.broadcast_to(jj * _c(8) + _c(u), (16,))
                val = plsc.load_gather(acc_v, [rows, jv]) * dv2
                plsc.store_scatter(acc_v, [rows, jv], val)
            return c3
        _fori(H // 8, initj, 0)
        return c2
    _fori(ROWS_T // 16, initg, 0)

    def chunk(ci, c2):
        base = ci * _c(EC)
        pltpu.sync_copy(src_hbm.at[pl.ds(base, EC)], src_v)
        pltpu.sync_copy(dst_hbm.at[pl.ds(base, EC)], dst_v)

        def scan(k, off):
            sv = src_v[pl.ds(k * _c(16), 16)]
            tv = dst_v[pl.ds(k * _c(16), 16)]
            dloc = tv - jnp.broadcast_to(lo_node, (16,))
            msk = (dloc >= _cv(0)) & (dloc < _cv(ROWS_T))
            m01 = jnp.where(msk, _cv(1), _cv(0))
            nrm = plsc.load_gather(dinv_v, [sv]) * plsc.load_gather(dinv_v, [tv])
            pos = plsc.cumsum(m01) + jnp.broadcast_to(off, (16,)) - _cv(1)
            plsc.store_scatter(esrc_l, [pos], sv, mask=msk)
            plsc.store_scatter(edloc_l, [pos], dloc, mask=msk)
            plsc.store_scatter(enrm_l, [pos], nrm, mask=msk)
            return off + jnp.sum(m01, dtype=jnp.int32)
        cnt = _fori(EC // 16, scan, _c(0))

        nsub = lax.shift_right_logical(cnt + _c(63), _c(6))

        def sub(si, c3):
            sbase = si * _c(64)
            pltpu.async_copy(hw_hbm.at[esrc_l.at[pl.ds(sbase, 64)]],
                             stage_v, sem).wait()
            for g in range(4):
                lanes = sbase + _c(g * 16) + _iota16()
                lmsk = lanes < jnp.broadcast_to(cnt, (16,))
                dl = plsc.load_gather(edloc_l, [lanes])
                dl = jnp.where(lmsk, dl, _cv(0))
                nr = plsc.load_gather(enrm_l, [lanes])
                srows = _cv(g * 16) + _iota16()

                def jloop(jj, c4, _srows=srows, _dl=dl, _nr=nr, _lmsk=lmsk):
                    for u in range(8):
                        jv = jnp.broadcast_to(jj * _c(8) + _c(u), (16,))
                        val = plsc.load_gather(stage_v, [_srows, jv]) * _nr
                        plsc.addupdate_scatter(acc_v, [_dl, jv], val,
                                               mask=_lmsk)
                    return c4
                _fori(H // 8, jloop, 0)
            return c3
        lax.fori_loop(_c(0), nsub, sub, 0)
        return c2
    _fori(E // EC, chunk, 0)

    def ep(r, c2):
        for jc in range(H // 16):
            v = acc_v[r, pl.ds(jc * 16, 16)] + bias_v[pl.ds(jc * 16, 16)]
            acc_v[r, pl.ds(jc * 16, 16)] = jnp.maximum(
                v, jnp.zeros((16,), jnp.float32))
        return c2
    _fori(ROWS_T, ep, 0)
    pltpu.sync_copy(acc_v, out_hbm.at[pl.ds(lo_node, ROWS_T)])


def _spmm(hw, src, dst, dinv, bias):
    f = pl.kernel(
        _spmm_body,
        out_type=jax.ShapeDtypeStruct((NPAD, H), jnp.float32),
        mesh=_SC_MESH,
        compiler_params=pltpu.CompilerParams(needs_layout_passes=False),
        scratch_types=[
            pltpu.VMEM((NPAD,), jnp.float32),
            pltpu.VMEM((EC,), jnp.int32),
            pltpu.VMEM((EC,), jnp.int32),
            pltpu.VMEM((EC,), jnp.int32),
            pltpu.VMEM((EC,), jnp.int32),
            pltpu.VMEM((EC,), jnp.float32),
            pltpu.VMEM((64, H), jnp.float32),
            pltpu.VMEM((ROWS_T, H), jnp.float32),
            pltpu.VMEM((H,), jnp.float32),
            pltpu.SemaphoreType.DMA,
        ],
    )
    return f(hw, src, dst, dinv, bias)


def kernel(x, edge_index, batch, d, d_index, W1, b1, W2, b2, W3, b3,
           Wf1, bf1, Wf2, bf2, Wf3, bf3):
    out_dtype = jnp.result_type(x.dtype, W1.dtype)
    x = x.astype(jnp.float32)
    d = d.astype(jnp.float32)
    W1, b1, W2, b2, W3, b3 = (a.astype(jnp.float32) for a in (W1, b1, W2, b2, W3, b3))
    Wf1, bf1, Wf2, bf2, Wf3, bf3 = (a.astype(jnp.float32) for a in (Wf1, bf1, Wf2, bf2, Wf3, bf3))
    src = edge_index[0].astype(jnp.int32)
    dst = edge_index[1].astype(jnp.int32)
    batch32 = batch.astype(jnp.int32)
    row = d_index[0].astype(jnp.int32)
    col0 = d_index[1].astype(jnp.int32)

    degp = _deg(dst).reshape(32, NPAD)
    dinv = _dinv(degp)

    xpad = jnp.pad(x, ((0, NPAD - N), (0, 0)))

    def gcn(h, W, b):
        hw = _matmul(h, W)
        return _spmm(hw, src, dst, dinv, b)

    x1 = gcn(xpad, W1, b1)
    x2 = gcn(x1, W2, b2)
    x3 = gcn(x2, W3, b3)
    xc = jnp.concatenate([x1, x2, x3], axis=-1)

    batchp = jnp.pad(batch32, ((0, NPAD - N),))
    pmat = _pbuild(row, col0, d, batchp)

    wf3p = jnp.pad(Wf3, ((0, 0), (0, 128 - C)))
    bf3p = jnp.pad(bf3, ((0, 128 - C),))
    out = _head(pmat, xc, Wf1, bf1, Wf2, bf2, wf3p, bf3p)
    return out[:, :C].astype(out_dtype)


# speedup vs baseline: 1.8700x; 1.8700x over previous
import functools

import jax
import jax.numpy as jnp
from jax import lax
from jax.experimental import pallas as pl
from jax.experimental.pallas import tpu as pltpu
from jax.experimental.pallas import tpu_sc as plsc

N = 10000
E = 160000
F_IN = 256
H = 256
LEV = 2
R_ = 3
NS = (R_ - 1) * LEV + 1
B = 32
NNZ = 800000
C = 10
NPAD = 10240


def _i0():
    return jnp.asarray(0, jnp.int32)


def _mm_body(x_ref, w_ref, o_ref):
    o_ref[...] = jnp.dot(x_ref[...], w_ref[...],
                         preferred_element_type=jnp.float32)


def _matmul(x, w, bm=2048):
    M, K = x.shape
    F = w.shape[1]
    assert M % bm == 0
    return pl.pallas_call(
        _mm_body,
        grid=(M // bm,),
        in_specs=[pl.BlockSpec((bm, K), lambda i: (i, _i0())),
                  pl.BlockSpec((K, F), lambda i: (_i0(), _i0())),
        ],
        out_specs=pl.BlockSpec((bm, F), lambda i: (i, _i0())),
        out_shape=jax.ShapeDtypeStruct((M, F), jnp.float32),
    )(x, w)


def _head_body(pm_ref, xc_ref, wf1_ref, bf1_ref, wf2_ref, bf2_ref,
               wf3_ref, bf3_ref, o_ref):
    pooled = jnp.dot(pm_ref[...], xc_ref[...],
                     preferred_element_type=jnp.float32)
    g = pooled.reshape(B, NS * 3 * H)
    h = jax.nn.relu(jnp.dot(g, wf1_ref[...],
                            preferred_element_type=jnp.float32) + bf1_ref[...])
    h = jax.nn.relu(jnp.dot(h, wf2_ref[...],
                            preferred_element_type=jnp.float32) + bf2_ref[...])
    logits = jnp.dot(h, wf3_ref[...],
                     preferred_element_type=jnp.float32) + bf3_ref[...]
    colid = jax.lax.broadcasted_iota(jnp.int32, logits.shape, 1)
    masked = jnp.where(colid < C, logits, -jnp.inf)
    mx = jnp.max(masked, axis=-1, keepdims=True)
    lse = jnp.log(jnp.sum(jnp.where(colid < C, jnp.exp(masked - mx), 0.0),
                          axis=-1, keepdims=True)) + mx
    o_ref[...] = jnp.where(colid < C, masked - lse, 0.0)


def _head(pmat, xc, wf1, bf1, wf2, bf2, wf3p, bf3p):
    full = lambda shape: pl.BlockSpec(shape, lambda: tuple(_i0() for _ in shape))
    return pl.pallas_call(
        _head_body,
        in_specs=[full((B * NS, NPAD)), full((NPAD, 3 * H)),
                  full((NS * 3 * H, 3 * H)), full((3 * H,)),
                  full((3 * H, H)), full((H,)),
                  full((H, 128)), full((128,))],
        out_specs=full((B, 128)),
        out_shape=jax.ShapeDtypeStruct((B, 128), jnp.float32),
    )(pmat, xc, wf1, bf1, wf2, bf2, wf3p, bf3p)


_SC_MESH = plsc.VectorSubcoreMesh(core_axis_name="c", subcore_axis_name="s")
PB_CH = 2000
PROWS = 5


def _c(v):
    return jnp.asarray(v, jnp.int32)


def _fori(n, body, init=0):
    return lax.fori_loop(_c(0), _c(n), body, init)


def _cv(v):
    return jnp.full((16,), v, jnp.int32)


def _divmod_n(v):
    q = (jnp.where(v >= _cv(N), _cv(1), _cv(0))
         + jnp.where(v >= _cv(2 * N), _cv(1), _cv(0))
         + jnp.where(v >= _cv(3 * N), _cv(1), _cv(0))
         + jnp.where(v >= _cv(4 * N), _cv(1), _cv(0)))
    return q, v - q * _cv(N)


def _pbuild_body(pk_hbm, batch_hbm, p_hbm,
                 st0_v, st1_v, batch_v, acc_v, sem0, sem1):
    wid = lax.axis_index("s") * _c(2) + lax.axis_index("c")
    lo = wid * _c(PROWS)
    nch = NNZ // PB_CH

    def zrow(r, carry):
        def zcol(j, c2):
            acc_v[r, pl.ds(j * _c(16), 16)] = jnp.zeros((16,), jnp.float32)
            return c2
        return _fori(NPAD // 16, zcol, carry)
    _fori(8, zrow, 0)

    pltpu.sync_copy(batch_hbm, batch_v)

    def process(st_v):
        def inner(k, c2):
            off = k * _c(16)
            rv = st_v[0, pl.ds(off, 16)]
            cv = st_v[1, pl.ds(off, 16)]
            dv = plsc.bitcast(st_v[2, pl.ds(off, 16)], jnp.float32)
            s, n_ = _divmod_n(rv)
            _, c = _divmod_n(cv)
            b = plsc.load_gather(batch_v, [n_])
            local = b * _cv(NS) + s - jnp.broadcast_to(lo, (16,))
            msk = (local >= _cv(0)) & (local < _cv(PROWS))
            local = jnp.where(msk, local, _cv(0))
            plsc.addupdate_scatter(acc_v, [local, c], dv, mask=msk)
            return c2
        _fori(PB_CH // 16, inner, 0)

    pltpu.make_async_copy(pk_hbm.at[_c(0)], st0_v, sem0).start()

    def pair(i2, carry):
        ci0 = i2 * _c(2)
        pltpu.make_async_copy(pk_hbm.at[ci0 + _c(1)], st1_v, sem1).start()
        pltpu.make_async_copy(pk_hbm.at[ci0], st0_v, sem0).wait()
        process(st0_v)

        @pl.when(ci0 + _c(2) < _c(nch))
        def _():
            pltpu.make_async_copy(pk_hbm.at[ci0 + _c(2)], st0_v, sem0).start()
        pltpu.make_async_copy(pk_hbm.at[ci0 + _c(1)], st1_v, sem1).wait()
        process(st1_v)
        return carry
    _fori(nch // 2, pair, 0)

    pltpu.sync_copy(acc_v, p_hbm.at[wid])


def _pbuild(packed, batchp):
    f = pl.kernel(
        _pbuild_body,
        out_type=jax.ShapeDtypeStruct((32, 8, NPAD), jnp.float32),
        mesh=_SC_MESH,
        compiler_params=pltpu.CompilerParams(needs_layout_passes=False),
        scratch_types=[
            pltpu.VMEM((3, PB_CH), jnp.int32),
            pltpu.VMEM((3, PB_CH), jnp.int32),
            pltpu.VMEM((NPAD,), jnp.int32),
            pltpu.VMEM((8, NPAD), jnp.float32),
            pltpu.SemaphoreType.DMA,
            pltpu.SemaphoreType.DMA,
        ],
    )
    out3 = f(packed, batchp)
    return out3[:, :PROWS, :].reshape(B * NS, NPAD)


EC = 2000
SUB = 32
SUBSH = 5
ROWS_T = NPAD // 32


def _iota16():
    return lax.iota(jnp.int32, 16)


def _deg_body(dst_hbm, degp_hbm, dst_v, deg_v):
    wid = lax.axis_index("s") * _c(2) + lax.axis_index("c")

    def zcol(j, c2):
        deg_v[0, pl.ds(j * _c(16), 16)] = jnp.zeros((16,), jnp.float32)
        return c2
    _fori(NPAD // 16, zcol, 0)

    nch = E // EC

    def chunk(i, c2):
        ci = wid + i * _c(32)

        @pl.when(ci < _c(nch))
        def _():
            pltpu.sync_copy(dst_hbm.at[pl.ds(ci * _c(EC), EC)], dst_v)

            def inner(k, c3):
                tv = dst_v[pl.ds(k * _c(16), 16)]
                plsc.addupdate_scatter(deg_v, [_cv(0), tv],
                                       jnp.full((16,), 1.0, jnp.float32))
                return c3
            _fori(EC // 16, inner, 0)
        return c2
    _fori((nch + 31) // 32, chunk, 0)
    pltpu.sync_copy(deg_v, degp_hbm.at[wid])


def _deg(dst):
    f = pl.kernel(
        _deg_body,
        out_type=jax.ShapeDtypeStruct((32, 1, NPAD), jnp.float32),
        mesh=_SC_MESH,
        compiler_params=pltpu.CompilerParams(needs_layout_passes=False),
        scratch_types=[
            pltpu.VMEM((EC,), jnp.int32),
            pltpu.VMEM((1, NPAD), jnp.float32),
        ],
    )
    return f(dst)


def _dinv_body(degp_ref, o_ref):
    deg = jnp.sum(degp_ref[...], axis=0) + 1.0
    o_ref[...] = lax.rsqrt(jnp.maximum(deg, 1.0))


def _dinv(degp):
    full = lambda shape: pl.BlockSpec(shape, lambda: tuple(_i0() for _ in shape))
    return pl.pallas_call(
        _dinv_body,
        in_specs=[full((32, NPAD))],
        out_specs=full((NPAD,)),
        out_shape=jax.ShapeDtypeStruct((NPAD,), jnp.float32),
    )(degp)


def _spmm_body(hw_hbm, epk_hbm, dinv_hbm, bias_hbm, out_hbm,
               dinv_v, est_v, esrc_l, edloc_l, enrm_l,
               stage0_v, stage1_v, acc_v, bias_v, semg0, semg1):
    wid = lax.axis_index("s") * _c(2) + lax.axis_index("c")
    lo_node = wid * _c(ROWS_T)

    pltpu.sync_copy(dinv_hbm, dinv_v.at[pl.ds(0, NPAD)])
    pltpu.sync_copy(bias_hbm, bias_v)

    def zpad(j, c2):
        dinv_v[pl.ds(_c(NPAD + j * 16), 16)] = jnp.zeros((16,), jnp.float32)
        return c2
    _fori(2, zpad, 0)

    def zl(k, c2):
        esrc_l[pl.ds(k * _c(16), 16)] = jnp.zeros((16,), jnp.int32)
        return c2
    _fori((EC + SUB) // 16, zl, 0)

    pltpu.sync_copy(hw_hbm.at[pl.ds(lo_node, ROWS_T)],
                    acc_v.at[pl.ds(0, ROWS_T)])

    def initr(r, c2):
        dv = dinv_v[pl.ds(lo_node + r, 16)][0]
        d2 = jnp.broadcast_to(dv * dv, (16,))
        for jc in range(H // 16):
            acc_v[r, pl.ds(jc * 16, 16)] = acc_v[r, pl.ds(jc * 16, 16)] * d2
        return c2
    _fori(ROWS_T, initr, 0)

    nch = E // EC

    def process_sub(stage_v, sbase):
        def egrp(g, c4):
            gl = g * _c(16)
            dlv = edloc_l[pl.ds(sbase + gl, 16)]
            nrv = enrm_l[pl.ds(sbase + gl, 16)]
            for u in range(16):
                dl = dlv[u]
                nr = jnp.broadcast_to(nrv[u], (16,))
                sr = gl + _c(u)
                for jc in range(H // 16):
                    val = stage_v[sr, pl.ds(jc * 16, 16)] * nr
                    plsc.addupdate(acc_v.at[dl, pl.ds(jc * 16, 16)], val)
            return c4
        _fori(SUB // 16, egrp, 0)

    def chunk(ci, c2):
        pltpu.sync_copy(epk_hbm.at[ci], est_v)

        def scan(k, off):
            sv = est_v[0, pl.ds(k * _c(16), 16)]
            tv = est_v[1, pl.ds(k * _c(16), 16)]
            dloc = tv - jnp.broadcast_to(lo_node, (16,))
            msk = (dloc >= _cv(0)) & (dloc < _cv(ROWS_T))
            m01 = jnp.where(msk, _cv(1), _cv(0))
            nrm = plsc.load_gather(dinv_v, [sv]) * plsc.load_gather(dinv_v, [tv])
            pos = plsc.cumsum(m01) + jnp.broadcast_to(off, (16,)) - _cv(1)
            plsc.store_scatter(esrc_l, [pos], sv, mask=msk)
            plsc.store_scatter(edloc_l, [pos], dloc, mask=msk)
            plsc.store_scatter(enrm_l, [pos], nrm, mask=msk)
            return off + jnp.sum(m01, dtype=jnp.int32)
        cnt = _fori(EC // 16, scan, _c(0))

        cntp = lax.shift_left(lax.shift_right_logical(cnt + _c(SUB - 1),
                                                      _c(SUBSH)), _c(SUBSH))
        for k in range(SUB // 16):
            pos = jnp.broadcast_to(cnt, (16,)) + _cv(k * 16) + _iota16()
            pm = pos < jnp.broadcast_to(cntp, (16,))
            plsc.store_scatter(esrc_l, [pos], _cv(0), mask=pm)
            plsc.store_scatter(edloc_l, [pos], _cv(ROWS_T), mask=pm)
            plsc.store_scatter(enrm_l, [pos], jnp.zeros((16,), jnp.float32),
                               mask=pm)

        nsub = lax.shift_right_logical(cntp, _c(SUBSH))
        npair = lax.shift_right_logical(nsub + _c(1), _c(1))

        @pl.when(nsub > _c(0))
        def _():
            pltpu.make_async_copy(hw_hbm.at[esrc_l.at[pl.ds(_c(0), SUB)]],
                                  stage0_v, semg0).start()

            def gpair(p, c3):
                s0 = p * _c(2 * SUB)
                s1 = s0 + _c(SUB)

                @pl.when(s1 < cntp)
                def _():
                    pltpu.make_async_copy(
                        hw_hbm.at[esrc_l.at[pl.ds(s1, SUB)]],
                        stage1_v, semg1).start()
                pltpu.make_async_copy(hw_hbm.at[esrc_l.at[pl.ds(s0, SUB)]],
                                      stage0_v, semg0).wait()
                process_sub(stage0_v, s0)

                @pl.when(s1 + _c(SUB) < cntp)
                def _():
                    pltpu.make_async_copy(
                        hw_hbm.at[esrc_l.at[pl.ds(s1 + _c(SUB), SUB)]],
                        stage0_v, semg0).start()

                @pl.when(s1 < cntp)
                def _():
                    pltpu.make_async_copy(
                        hw_hbm.at[esrc_l.at[pl.ds(s1, SUB)]],
                        stage1_v, semg1).wait()
                    process_sub(stage1_v, s1)
                return c3
            lax.fori_loop(_c(0), npair, gpair, 0)
        return c2
    _fori(nch, chunk, 0)

    def ep(r, c2):
        for jc in range(H // 16):
            v = acc_v[r, pl.ds(jc * 16, 16)] + bias_v[pl.ds(jc * 16, 16)]
            acc_v[r, pl.ds(jc * 16, 16)] = jnp.maximum(
                v, jnp.zeros((16,), jnp.float32))
        return c2
    _fori(ROWS_T, ep, 0)
    pltpu.sync_copy(acc_v.at[pl.ds(0, ROWS_T)],
                    out_hbm.at[pl.ds(lo_node, ROWS_T)])


def _spmm(hw, epk, dinv, bias):
    f = pl.kernel(
        _spmm_body,
        out_type=jax.ShapeDtypeStruct((NPAD, H), jnp.float32),
        mesh=_SC_MESH,
        compiler_params=pltpu.CompilerParams(needs_layout_passes=False),
        scratch_types=[
            pltpu.VMEM((NPAD + 32,), jnp.float32),
            pltpu.VMEM((2, EC), jnp.int32),
            pltpu.VMEM((EC + SUB,), jnp.int32),
            pltpu.VMEM((EC + SUB,), jnp.int32),
            pltpu.VMEM((EC + SUB,), jnp.float32),
            pltpu.VMEM((SUB, H), jnp.float32),
            pltpu.VMEM((SUB, H), jnp.float32),
            pltpu.VMEM((ROWS_T + 8, H), jnp.float32),
            pltpu.VMEM((H,), jnp.float32),
            pltpu.SemaphoreType.DMA,
            pltpu.SemaphoreType.DMA,
        ],
    )
    return f(hw, epk, dinv, bias)


def kernel(x, edge_index, batch, d, d_index, W1, b1, W2, b2, W3, b3,
           Wf1, bf1, Wf2, bf2, Wf3, bf3):
    out_dtype = jnp.result_type(x.dtype, W1.dtype)
    x = x.astype(jnp.float32)
    d = d.astype(jnp.float32)
    W1, b1, W2, b2, W3, b3 = (a.astype(jnp.float32) for a in (W1, b1, W2, b2, W3, b3))
    Wf1, bf1, Wf2, bf2, Wf3, bf3 = (a.astype(jnp.float32) for a in (Wf1, bf1, Wf2, bf2, Wf3, bf3))
    src = edge_index[0].astype(jnp.int32)
    dst = edge_index[1].astype(jnp.int32)
    batch32 = batch.astype(jnp.int32)
    row = d_index[0].astype(jnp.int32)
    col0 = d_index[1].astype(jnp.int32)

    degp = _deg(dst).reshape(32, NPAD)
    dinv = _dinv(degp)

    epk = jnp.stack([src.reshape(E // EC, EC), dst.reshape(E // EC, EC)],
                    axis=1)

    xpad = jnp.pad(x, ((0, NPAD - N), (0, 0)))

    def gcn(h, W, b):
        hw = _matmul(h, W)
        return _spmm(hw, epk, dinv, b)

    x1 = gcn(xpad, W1, b1)
    x2 = gcn(x1, W2, b2)
    x3 = gcn(x2, W3, b3)
    xc = jnp.concatenate([x1, x2, x3], axis=-1)

    batchp = jnp.pad(batch32, ((0, NPAD - N),))
    d_i32 = jax.lax.bitcast_convert_type(d, jnp.int32)
    packed = jnp.stack([row.reshape(NNZ // PB_CH, PB_CH),
                        col0.reshape(NNZ // PB_CH, PB_CH),
                        d_i32.reshape(NNZ // PB_CH, PB_CH)], axis=1)
    pmat = _pbuild(packed, batchp)

    wf3p = jnp.pad(Wf3, ((0, 0), (0, 128 - C)))
    bf3p = jnp.pad(bf3, ((0, 128 - C),))
    out = _head(pmat, xc, Wf1, bf1, Wf2, bf2, wf3p, bf3p)
    return out[:, :C].astype(out_dtype)


# speedup vs baseline: 1.8966x; 1.0142x over previous
import functools

import jax
import jax.numpy as jnp
from jax import lax
from jax.experimental import pallas as pl
from jax.experimental.pallas import tpu as pltpu
from jax.experimental.pallas import tpu_sc as plsc

N = 10000
E = 160000
F_IN = 256
H = 256
LEV = 2
R_ = 3
NS = (R_ - 1) * LEV + 1
B = 32
NNZ = 800000
C = 10
NPAD = 10240


def _i0():
    return jnp.asarray(0, jnp.int32)


def _mm_body(x_ref, w_ref, o_ref):
    o_ref[...] = jnp.dot(x_ref[...], w_ref[...],
                         preferred_element_type=jnp.float32)


def _matmul(x, w, bm=2048):
    M, K = x.shape
    F = w.shape[1]
    assert M % bm == 0
    return pl.pallas_call(
        _mm_body,
        grid=(M // bm,),
        in_specs=[pl.BlockSpec((bm, K), lambda i: (i, _i0())),
                  pl.BlockSpec((K, F), lambda i: (_i0(), _i0())),
        ],
        out_specs=pl.BlockSpec((bm, F), lambda i: (i, _i0())),
        out_shape=jax.ShapeDtypeStruct((M, F), jnp.float32),
    )(x, w)


def _head_body(pm_ref, xc_ref, wf1_ref, bf1_ref, wf2_ref, bf2_ref,
               wf3_ref, bf3_ref, o_ref):
    pooled = jnp.dot(pm_ref[...], xc_ref[...],
                     preferred_element_type=jnp.float32)
    g = pooled.reshape(B, NS * 3 * H)
    h = jax.nn.relu(jnp.dot(g, wf1_ref[...],
                            preferred_element_type=jnp.float32) + bf1_ref[...])
    h = jax.nn.relu(jnp.dot(h, wf2_ref[...],
                            preferred_element_type=jnp.float32) + bf2_ref[...])
    logits = jnp.dot(h, wf3_ref[...],
                     preferred_element_type=jnp.float32) + bf3_ref[...]
    colid = jax.lax.broadcasted_iota(jnp.int32, logits.shape, 1)
    masked = jnp.where(colid < C, logits, -jnp.inf)
    mx = jnp.max(masked, axis=-1, keepdims=True)
    lse = jnp.log(jnp.sum(jnp.where(colid < C, jnp.exp(masked - mx), 0.0),
                          axis=-1, keepdims=True)) + mx
    o_ref[...] = jnp.where(colid < C, masked - lse, 0.0)


def _head(pmat, xc, wf1, bf1, wf2, bf2, wf3p, bf3p):
    full = lambda shape: pl.BlockSpec(shape, lambda: tuple(_i0() for _ in shape))
    return pl.pallas_call(
        _head_body,
        in_specs=[full((B * NS, NPAD)), full((NPAD, 3 * H)),
                  full((NS * 3 * H, 3 * H)), full((3 * H,)),
                  full((3 * H, H)), full((H,)),
                  full((H, 128)), full((128,))],
        out_specs=full((B, 128)),
        out_shape=jax.ShapeDtypeStruct((B, 128), jnp.float32),
    )(pmat, xc, wf1, bf1, wf2, bf2, wf3p, bf3p)


_SC_MESH = plsc.VectorSubcoreMesh(core_axis_name="c", subcore_axis_name="s")
PB_CH = 2000
PROWS = 5


def _c(v):
    return jnp.asarray(v, jnp.int32)


def _fori(n, body, init=0):
    return lax.fori_loop(_c(0), _c(n), body, init)


def _cv(v):
    return jnp.full((16,), v, jnp.int32)


def _divmod_n(v):
    q = (jnp.where(v >= _cv(N), _cv(1), _cv(0))
         + jnp.where(v >= _cv(2 * N), _cv(1), _cv(0))
         + jnp.where(v >= _cv(3 * N), _cv(1), _cv(0))
         + jnp.where(v >= _cv(4 * N), _cv(1), _cv(0)))
    return q, v - q * _cv(N)


def _pbuild_body(pk_hbm, batch_hbm, p_hbm,
                 st0_v, st1_v, batch_v, acc_v, sem0, sem1):
    wid = lax.axis_index("s") * _c(2) + lax.axis_index("c")
    lo = wid * _c(PROWS)
    nch = NNZ // PB_CH

    def zrow(r, carry):
        def zcol(j, c2):
            acc_v[r, pl.ds(j * _c(16), 16)] = jnp.zeros((16,), jnp.float32)
            return c2
        return _fori(NPAD // 16, zcol, carry)
    _fori(8, zrow, 0)

    pltpu.sync_copy(batch_hbm, batch_v)

    def process(st_v):
        def inner(k, c2):
            for u in range(5):
                off = k * _c(80) + _c(u * 16)
                rv = st_v[0, pl.ds(off, 16)]
                cv = st_v[1, pl.ds(off, 16)]
                dv = plsc.bitcast(st_v[2, pl.ds(off, 16)], jnp.float32)
                s, n_ = _divmod_n(rv)
                _, c = _divmod_n(cv)
                b = plsc.load_gather(batch_v, [n_])
                local = b * _cv(NS) + s - jnp.broadcast_to(lo, (16,))
                msk = (local >= _cv(0)) & (local < _cv(PROWS))
                local = jnp.where(msk, local, _cv(0))
                plsc.addupdate_scatter(acc_v, [local, c], dv, mask=msk)
            return c2
        _fori(PB_CH // 80, inner, 0)

    pltpu.make_async_copy(pk_hbm.at[_c(0)], st0_v, sem0).start()

    def pair(i2, carry):
        ci0 = i2 * _c(2)
        pltpu.make_async_copy(pk_hbm.at[ci0 + _c(1)], st1_v, sem1).start()
        pltpu.make_async_copy(pk_hbm.at[ci0], st0_v, sem0).wait()
        process(st0_v)

        @pl.when(ci0 + _c(2) < _c(nch))
        def _():
            pltpu.make_async_copy(pk_hbm.at[ci0 + _c(2)], st0_v, sem0).start()
        pltpu.make_async_copy(pk_hbm.at[ci0 + _c(1)], st1_v, sem1).wait()
        process(st1_v)
        return carry
    _fori(nch // 2, pair, 0)

    pltpu.sync_copy(acc_v, p_hbm.at[wid])


def _pbuild(packed, batchp):
    f = pl.kernel(
        _pbuild_body,
        out_type=jax.ShapeDtypeStruct((32, 8, NPAD), jnp.float32),
        mesh=_SC_MESH,
        compiler_params=pltpu.CompilerParams(needs_layout_passes=False),
        scratch_types=[
            pltpu.VMEM((3, PB_CH), jnp.int32),
            pltpu.VMEM((3, PB_CH), jnp.int32),
            pltpu.VMEM((NPAD,), jnp.int32),
            pltpu.VMEM((8, NPAD), jnp.float32),
            pltpu.SemaphoreType.DMA,
            pltpu.SemaphoreType.DMA,
        ],
    )
    out3 = f(packed, batchp)
    return out3[:, :PROWS, :].reshape(B * NS, NPAD)


EC = 2000
SUB = 32
SUBSH = 5
ROWS_T = NPAD // 32


def _iota16():
    return lax.iota(jnp.int32, 16)


def _deg_body(dst_hbm, degp_hbm, dst_v, deg_v):
    wid = lax.axis_index("s") * _c(2) + lax.axis_index("c")

    def zcol(j, c2):
        deg_v[0, pl.ds(j * _c(16), 16)] = jnp.zeros((16,), jnp.float32)
        return c2
    _fori(NPAD // 16, zcol, 0)

    nch = E // EC

    def chunk(i, c2):
        ci = wid + i * _c(32)

        @pl.when(ci < _c(nch))
        def _():
            pltpu.sync_copy(dst_hbm.at[pl.ds(ci * _c(EC), EC)], dst_v)

            def inner(k, c3):
                tv = dst_v[pl.ds(k * _c(16), 16)]
                plsc.addupdate_scatter(deg_v, [_cv(0), tv],
                                       jnp.full((16,), 1.0, jnp.float32))
                return c3
            _fori(EC // 16, inner, 0)
        return c2
    _fori((nch + 31) // 32, chunk, 0)
    pltpu.sync_copy(deg_v, degp_hbm.at[wid])


def _deg(dst):
    f = pl.kernel(
        _deg_body,
        out_type=jax.ShapeDtypeStruct((32, 1, NPAD), jnp.float32),
        mesh=_SC_MESH,
        compiler_params=pltpu.CompilerParams(needs_layout_passes=False),
        scratch_types=[
            pltpu.VMEM((EC,), jnp.int32),
            pltpu.VMEM((1, NPAD), jnp.float32),
        ],
    )
    return f(dst)


def _dinv_body(degp_ref, o_ref):
    deg = jnp.sum(degp_ref[...], axis=0) + 1.0
    o_ref[...] = lax.rsqrt(jnp.maximum(deg, 1.0))


def _dinv(degp):
    full = lambda shape: pl.BlockSpec(shape, lambda: tuple(_i0() for _ in shape))
    return pl.pallas_call(
        _dinv_body,
        in_specs=[full((32, NPAD))],
        out_specs=full((NPAD,)),
        out_shape=jax.ShapeDtypeStruct((NPAD,), jnp.float32),
    )(degp)


def _spmm_body(hw_hbm, epk_hbm, dinv_hbm, bias_hbm, out_hbm,
               dinv_v, est_v, esrc_l, edloc_l, enrm_l,
               stage0_v, stage1_v, acc_v, bias_v, semg0, semg1):
    wid = lax.axis_index("s") * _c(2) + lax.axis_index("c")
    lo_node = wid * _c(ROWS_T)

    pltpu.sync_copy(dinv_hbm, dinv_v.at[pl.ds(0, NPAD)])
    pltpu.sync_copy(bias_hbm, bias_v)

    def zpad(j, c2):
        dinv_v[pl.ds(_c(NPAD + j * 16), 16)] = jnp.zeros((16,), jnp.float32)
        return c2
    _fori(2, zpad, 0)

    def zl(k, c2):
        esrc_l[pl.ds(k * _c(16), 16)] = jnp.zeros((16,), jnp.int32)
        return c2
    _fori((EC + SUB) // 16, zl, 0)

    pltpu.sync_copy(hw_hbm.at[pl.ds(lo_node, ROWS_T)],
                    acc_v.at[pl.ds(0, ROWS_T)])

    def initr(r, c2):
        dv = dinv_v[pl.ds(lo_node + r, 16)][0]
        d2 = jnp.broadcast_to(dv * dv, (16,))
        for jc in range(H // 16):
            acc_v[r, pl.ds(jc * 16, 16)] = acc_v[r, pl.ds(jc * 16, 16)] * d2
        return c2
    _fori(ROWS_T, initr, 0)

    nch = E // EC

    def process_sub(stage_v, sbase):
        def egrp(g, c4):
            gl = g * _c(16)
            dlv = edloc_l[pl.ds(sbase + gl, 16)]
            nrv = enrm_l[pl.ds(sbase + gl, 16)]
            for u in range(16):
                dl = dlv[u]
                nr = jnp.broadcast_to(nrv[u], (16,))
                sr = gl + _c(u)
                for jc in range(H // 16):
                    val = stage_v[sr, pl.ds(jc * 16, 16)] * nr
                    plsc.addupdate(acc_v.at[dl, pl.ds(jc * 16, 16)], val)
            return c4
        _fori(SUB // 16, egrp, 0)

    def chunk(ci, c2):
        pltpu.sync_copy(epk_hbm.at[ci], est_v)

        def scan(k, off):
            o = off
            for u in range(5):
                kb = k * _c(80) + _c(u * 16)
                sv = est_v[0, pl.ds(kb, 16)]
                tv = est_v[1, pl.ds(kb, 16)]
                dloc = tv - jnp.broadcast_to(lo_node, (16,))
                msk = (dloc >= _cv(0)) & (dloc < _cv(ROWS_T))
                m01 = jnp.where(msk, _cv(1), _cv(0))
                nrm = (plsc.load_gather(dinv_v, [sv])
                       * plsc.load_gather(dinv_v, [tv]))
                inc = plsc.cumsum(m01)
                pos = inc + jnp.broadcast_to(o, (16,)) - _cv(1)
                plsc.store_scatter(esrc_l, [pos], sv, mask=msk)
                plsc.store_scatter(edloc_l, [pos], dloc, mask=msk)
                plsc.store_scatter(enrm_l, [pos], nrm, mask=msk)
                o = o + inc[15]
            return o
        cnt = _fori(EC // 80, scan, _c(0))

        cntp = lax.shift_left(lax.shift_right_logical(cnt + _c(SUB - 1),
                                                      _c(SUBSH)), _c(SUBSH))
        for k in range(SUB // 16):
            pos = jnp.broadcast_to(cnt, (16,)) + _cv(k * 16) + _iota16()
            pm = pos < jnp.broadcast_to(cntp, (16,))
            plsc.store_scatter(esrc_l, [pos], _cv(0), mask=pm)
            plsc.store_scatter(edloc_l, [pos], _cv(ROWS_T), mask=pm)
            plsc.store_scatter(enrm_l, [pos], jnp.zeros((16,), jnp.float32),
                               mask=pm)

        nsub = lax.shift_right_logical(cntp, _c(SUBSH))
        npair = lax.shift_right_logical(nsub + _c(1), _c(1))

        @pl.when(nsub > _c(0))
        def _():
            pltpu.make_async_copy(hw_hbm.at[esrc_l.at[pl.ds(_c(0), SUB)]],
                                  stage0_v, semg0).start()

            def gpair(p, c3):
                s0 = p * _c(2 * SUB)
                s1 = s0 + _c(SUB)

                @pl.when(s1 < cntp)
                def _():
                    pltpu.make_async_copy(
                        hw_hbm.at[esrc_l.at[pl.ds(s1, SUB)]],
                        stage1_v, semg1).start()
                pltpu.make_async_copy(hw_hbm.at[esrc_l.at[pl.ds(s0, SUB)]],
                                      stage0_v, semg0).wait()
                process_sub(stage0_v, s0)

                @pl.when(s1 + _c(SUB) < cntp)
                def _():
                    pltpu.make_async_copy(
                        hw_hbm.at[esrc_l.at[pl.ds(s1 + _c(SUB), SUB)]],
                        stage0_v, semg0).start()

                @pl.when(s1 < cntp)
                def _():
                    pltpu.make_async_copy(
                        hw_hbm.at[esrc_l.at[pl.ds(s1, SUB)]],
                        stage1_v, semg1).wait()
                    process_sub(stage1_v, s1)
                return c3
            lax.fori_loop(_c(0), npair, gpair, 0)
        return c2
    _fori(nch, chunk, 0)

    def ep(r, c2):
        for jc in range(H // 16):
            v = acc_v[r, pl.ds(jc * 16, 16)] + bias_v[pl.ds(jc * 16, 16)]
            acc_v[r, pl.ds(jc * 16, 16)] = jnp.maximum(
                v, jnp.zeros((16,), jnp.float32))
        return c2
    _fori(ROWS_T, ep, 0)
    pltpu.sync_copy(acc_v.at[pl.ds(0, ROWS_T)],
                    out_hbm.at[pl.ds(lo_node, ROWS_T)])


def _spmm(hw, epk, dinv, bias):
    f = pl.kernel(
        _spmm_body,
        out_type=jax.ShapeDtypeStruct((NPAD, H), jnp.float32),
        mesh=_SC_MESH,
        compiler_params=pltpu.CompilerParams(needs_layout_passes=False),
        scratch_types=[
            pltpu.VMEM((NPAD + 32,), jnp.float32),
            pltpu.VMEM((2, EC), jnp.int32),
            pltpu.VMEM((EC + SUB,), jnp.int32),
            pltpu.VMEM((EC + SUB,), jnp.int32),
            pltpu.VMEM((EC + SUB,), jnp.float32),
            pltpu.VMEM((SUB, H), jnp.float32),
            pltpu.VMEM((SUB, H), jnp.float32),
            pltpu.VMEM((ROWS_T + 8, H), jnp.float32),
            pltpu.VMEM((H,), jnp.float32),
            pltpu.SemaphoreType.DMA,
            pltpu.SemaphoreType.DMA,
        ],
    )
    return f(hw, epk, dinv, bias)


def kernel(x, edge_index, batch, d, d_index, W1, b1, W2, b2, W3, b3,
           Wf1, bf1, Wf2, bf2, Wf3, bf3):
    out_dtype = jnp.result_type(x.dtype, W1.dtype)
    x = x.astype(jnp.float32)
    d = d.astype(jnp.float32)
    W1, b1, W2, b2, W3, b3 = (a.astype(jnp.float32) for a in (W1, b1, W2, b2, W3, b3))
    Wf1, bf1, Wf2, bf2, Wf3, bf3 = (a.astype(jnp.float32) for a in (Wf1, bf1, Wf2, bf2, Wf3, bf3))
    src = edge_index[0].astype(jnp.int32)
    dst = edge_index[1].astype(jnp.int32)
    batch32 = batch.astype(jnp.int32)
    row = d_index[0].astype(jnp.int32)
    col0 = d_index[1].astype(jnp.int32)

    degp = _deg(dst).reshape(32, NPAD)
    dinv = _dinv(degp)

    epk = jnp.stack([src.reshape(E // EC, EC), dst.reshape(E // EC, EC)],
                    axis=1)

    xpad = jnp.pad(x, ((0, NPAD - N), (0, 0)))

    def gcn(h, W, b):
        hw = _matmul(h, W)
        return _spmm(hw, epk, dinv, b)

    x1 = gcn(xpad, W1, b1)
    x2 = gcn(x1, W2, b2)
    x3 = gcn(x2, W3, b3)
    xc = jnp.concatenate([x1, x2, x3], axis=-1)

    batchp = jnp.pad(batch32, ((0, NPAD - N),))
    d_i32 = jax.lax.bitcast_convert_type(d, jnp.int32)
    packed = jnp.stack([row.reshape(NNZ // PB_CH, PB_CH),
                        col0.reshape(NNZ // PB_CH, PB_CH),
                        d_i32.reshape(NNZ // PB_CH, PB_CH)], axis=1)
    pmat = _pbuild(packed, batchp)

    wf3p = jnp.pad(Wf3, ((0, 0), (0, 128 - C)))
    bf3p = jnp.pad(bf3, ((0, 128 - C),))
    out = _head(pmat, xc, Wf1, bf1, Wf2, bf2, wf3p, bf3p)
    return out[:, :C].astype(out_dtype)
